# baseline (speedup 1.0000x reference)
import functools

import jax
import jax.numpy as jnp
from jax import lax
from jax.experimental import pallas as pl
from jax.experimental.pallas import tpu as pltpu
from jax.experimental.pallas import tpu_sc as plsc

N = 100000
IN_CH = 128
H = 64
C = 10000

NC = 2
NS = 16
NW = NC * NS

N_PAD = 102400
RPW = N_PAD // NW
GCH = 400
CPT = 320
C_PAD = CPT * NW
SCH = 512
RB = 512
NEG = -3.0e38


def _bounds_body(cl_ref, out_ref):
    arr = cl_ref[...]
    acc = jnp.zeros((8, 128), jnp.int32)
    pos = lax.broadcasted_iota(jnp.int32, (8, 128), 0) * 128 + \
        lax.broadcasted_iota(jnp.int32, (8, 128), 1)
    for t in range(NW + 1):
        thr = min(CPT * t, C)
        cnt = jnp.sum((arr < thr).astype(jnp.int32))
        acc = jnp.where(pos == t, cnt, acc)
    out_ref[...] = acc


def _compute_bounds(cl2d):
    return pl.pallas_call(
        _bounds_body,
        out_shape=jax.ShapeDtypeStruct((8, 128), jnp.int32),
    )(cl2d)


def _ln(u, g, b):
    mu = jnp.mean(u, axis=-1, keepdims=True)
    var = jnp.mean((u - mu) ** 2, axis=-1, keepdims=True)
    return (u - mu) * lax.rsqrt(var + 1e-5) * g + b


def _mlp_tail(u, pv, w2_ref):
    u = jnp.maximum(_ln(u, pv[1:2, :], pv[2:3, :]), 0.0)
    v = jnp.dot(u, w2_ref[...], preferred_element_type=jnp.float32) + pv[3:4, :]
    return jnp.maximum(_ln(v, pv[4:5, :], pv[5:6, :]), 0.0)


def _mlp0_body(x_ref, w1_ref, w2_ref, pv_ref, out_ref):
    pv = pv_ref[...]
    u = jnp.dot(x_ref[...], w1_ref[...], preferred_element_type=jnp.float32)
    out_ref[...] = _mlp_tail(u + pv[0:1, :], pv, w2_ref)


def _mlp0(x, w1, w2, pv):
    return pl.pallas_call(
        _mlp0_body,
        grid=(N_PAD // RB,),
        in_specs=[
            pl.BlockSpec((RB, IN_CH), lambda i: (i, 0)),
            pl.BlockSpec((IN_CH, H), lambda i: (0, 0)),
            pl.BlockSpec((H, H), lambda i: (0, 0)),
            pl.BlockSpec((8, H), lambda i: (0, 0)),
        ],
        out_specs=pl.BlockSpec((RB, H), lambda i: (i, 0)),
        out_shape=jax.ShapeDtypeStruct((N_PAD, H), jnp.float32),
    )(x, w1, w2, pv)


def _layer_body(h_ref, g_ref, w1t_ref, w1b_ref, w2_ref, pv_ref, out_ref):
    pv = pv_ref[...]
    u = jnp.dot(h_ref[...], w1t_ref[...], preferred_element_type=jnp.float32)
    u = u + jnp.dot(g_ref[:, :H], w1b_ref[...],
                    preferred_element_type=jnp.float32)
    out_ref[...] = _mlp_tail(u + pv[0:1, :], pv, w2_ref)


def _layer(h, g, w1t, w1b, w2, pv):
    return pl.pallas_call(
        _layer_body,
        grid=(N_PAD // RB,),
        in_specs=[
            pl.BlockSpec((RB, H), lambda i: (i, 0)),
            pl.BlockSpec((RB, 2 * H), lambda i: (i, 0)),
            pl.BlockSpec((H, H), lambda i: (0, 0)),
            pl.BlockSpec((H, H), lambda i: (0, 0)),
            pl.BlockSpec((H, H), lambda i: (0, 0)),
            pl.BlockSpec((8, H), lambda i: (0, 0)),
        ],
        out_specs=pl.BlockSpec((RB, H), lambda i: (i, 0)),
        out_shape=jax.ShapeDtypeStruct((N_PAD, H), jnp.float32),
    )(h, g, w1t, w1b, w2, pv)


def _finalpre_body(h_ref, g_ref, wt_ref, wb_ref, pv_ref, out_ref):
    u = jnp.dot(h_ref[...], wt_ref[...], preferred_element_type=jnp.float32)
    u = u + jnp.dot(g_ref[:, :H], wb_ref[...],
                    preferred_element_type=jnp.float32)
    out_ref[...] = u + pv_ref[0:1, :]


def _finalpre(h, g, wlt, wlb, pv):
    return pl.pallas_call(
        _finalpre_body,
        grid=(N_PAD // RB,),
        in_specs=[
            pl.BlockSpec((RB, H), lambda i: (i, 0)),
            pl.BlockSpec((RB, 2 * H), lambda i: (i, 0)),
            pl.BlockSpec((H, H), lambda i: (0, 0)),
            pl.BlockSpec((H, H), lambda i: (0, 0)),
            pl.BlockSpec((8, H), lambda i: (0, 0)),
        ],
        out_specs=pl.BlockSpec((RB, H), lambda i: (i, 0)),
        out_shape=jax.ShapeDtypeStruct((N_PAD, H), jnp.float32),
    )(h, g, wlt, wlb, pv)


def _norm_body(z_ref, out_ref):
    z = z_ref[:, :H]
    s = jnp.sum(z * z, axis=-1, keepdims=True)
    out_ref[...] = z * lax.rsqrt(jnp.maximum(s, 1e-24))


def _norm(z):
    return pl.pallas_call(
        _norm_body,
        out_shape=jax.ShapeDtypeStruct((C_PAD, H), jnp.float32),
    )(z)


@functools.lru_cache(maxsize=None)
def _sc_mesh():
    return plsc.VectorSubcoreMesh(core_axis_name="c", subcore_axis_name="s",
                                  num_cores=NC, num_subcores=NS)


def _segmax_sc_body(neg_init, h_hbm, cl_hbm, bounds_hbm, xm_hbm,
                    bounds_s, cl_v, h_v, acc_v, sem):
    wid = lax.axis_index("s") * NC + lax.axis_index("c")
    c0 = wid * CPT
    init = NEG if neg_init else 0.0
    negv = jnp.full((16,), NEG, jnp.float32)
    zero16f = jnp.zeros((16,), jnp.float32)
    initv = jnp.full((16,), init, jnp.float32)

    pltpu.sync_copy(bounds_hbm.at[pl.ds(0, 64)], bounds_s)
    rs = bounds_s[pl.ds(wid, 16)][0]
    re = bounds_s[pl.ds(wid + 1, 16)][0]
    base = (rs // SCH) * SCH
    nchunks = (re - base + (SCH - 1)) // SCH

    def zero_body(r, _):
        for q in range(2 * H // 16):
            acc_v[r, pl.ds(16 * q, 16)] = initv
        return 0
    lax.fori_loop(0, CPT, zero_body, 0)

    def chunk_body(k, _):
        start = base + k * SCH
        pltpu.sync_copy(h_hbm.at[pl.ds(start, SCH)], h_v)
        pltpu.sync_copy(cl_hbm.at[pl.ds(start, SCH)], cl_v.at[pl.ds(0, SCH)])
        lo = jnp.maximum(rs - start, 0)
        hi = jnp.minimum(re - start, SCH)

        def row_body(j, _):
            li = cl_v[pl.ds(j, 16)][0] - c0
            for q in range(H // 16):
                v = h_v[j, pl.ds(16 * q, 16)]
                a = acc_v[li, pl.ds(16 * q, 16)]
                acc_v[li, pl.ds(16 * q, 16)] = jnp.maximum(a, v)
            return 0
        lax.fori_loop(lo, hi, row_body, 0)
        return 0
    lax.fori_loop(0, nchunks, chunk_body, 0)

    if neg_init:
        def fix_body(r, _):
            for q in range(H // 16):
                a = acc_v[r, pl.ds(16 * q, 16)]
                acc_v[r, pl.ds(16 * q, 16)] = jnp.where(a <= negv, zero16f, a)
            return 0
        lax.fori_loop(0, CPT, fix_body, 0)

    pltpu.sync_copy(acc_v, xm_hbm.at[pl.ds(c0, CPT)])


def _segmax(h, cl, bounds, neg_init):
    return pl.kernel(
        functools.partial(_segmax_sc_body, neg_init),
        out_type=jax.ShapeDtypeStruct((C_PAD, 2 * H), jnp.float32),
        mesh=_sc_mesh(),
        compiler_params=pltpu.CompilerParams(needs_layout_passes=False),
        scratch_types=[
            pltpu.VMEM((64,), jnp.int32),
            pltpu.VMEM((SCH + 16,), jnp.int32),
            pltpu.VMEM((SCH, H), jnp.float32),
            pltpu.VMEM((CPT, 2 * H), jnp.float32),
            pltpu.SemaphoreType.DMA,
        ],
    )(h, cl, bounds)


def _gather_sc_body(m_hbm, cl_hbm, g_hbm, idx_v, rows0, rows1,
                    gs0, gs1, ss0, ss1):
    wid = lax.axis_index("s") * NC + lax.axis_index("c")
    base = wid * RPW
    nk = RPW // GCH
    bufs = (rows0, rows1)
    gsem = (gs0, gs1)
    ssem = (ss0, ss1)
    pltpu.sync_copy(cl_hbm.at[pl.ds(base, RPW)], idx_v)
    gd = [None] * nk
    sd = [None] * nk
    for k in range(nk):
        b = k & 1
        if k >= 2:
            sd[k - 2].wait()
        gd[k] = pltpu.async_copy(
            m_hbm.at[idx_v.at[pl.ds(k * GCH, GCH)]], bufs[b], gsem[b])
        if k >= 1:
            gd[k - 1].wait()
            sd[k - 1] = pltpu.async_copy(
                bufs[1 - b], g_hbm.at[pl.ds(base + (k - 1) * GCH, GCH)],
                ssem[1 - b])
    gd[nk - 1].wait()
    sd[nk - 1] = pltpu.async_copy(
        bufs[(nk - 1) & 1], g_hbm.at[pl.ds(base + (nk - 1) * GCH, GCH)],
        ssem[(nk - 1) & 1])
    sd[nk - 2].wait()
    sd[nk - 1].wait()


def _gather(m, cl):
    return pl.kernel(
        _gather_sc_body,
        out_type=jax.ShapeDtypeStruct((N_PAD, 2 * H), jnp.float32),
        mesh=_sc_mesh(),
        compiler_params=pltpu.CompilerParams(needs_layout_passes=False),
        scratch_types=[
            pltpu.VMEM((RPW,), jnp.int32),
            pltpu.VMEM((GCH, 2 * H), jnp.float32),
            pltpu.VMEM((GCH, 2 * H), jnp.float32),
            pltpu.SemaphoreType.DMA,
            pltpu.SemaphoreType.DMA,
            pltpu.SemaphoreType.DMA,
            pltpu.SemaphoreType.DMA,
        ],
    )(m, cl)


def kernel(x, cluster, W1_0, b1_0, g1_0, be1_0, W2_0, b2_0, g2_0, be2_0,
           W1_1, b1_1, g1_1, be1_1, W2_1, b2_1, g2_1, be2_1,
           W1_2, b1_2, g1_2, be1_2, W2_2, b2_2, g2_2, be2_2, Wl, bl):
    x_pad = jnp.pad(x, ((0, N_PAD - N), (0, 0)))
    cl_pad = jnp.pad(cluster, (0, N_PAD - N), constant_values=C)
    cl2d = cl_pad.reshape(N_PAD // 128, 128)
    bounds = _compute_bounds(cl2d).reshape(-1)

    zeros = jnp.zeros((H,), jnp.float32)

    def pvec(rows):
        rows = list(rows) + [zeros] * (8 - len(rows))
        return jnp.stack(rows)

    pv0 = pvec([b1_0, g1_0, be1_0, b2_0, g2_0, be2_0])
    pv1 = pvec([b1_1, g1_1, be1_1, b2_1, g2_1, be2_1])
    pv2 = pvec([b1_2, g1_2, be1_2, b2_2, g2_2, be2_2])
    pvl = pvec([bl])

    h = _mlp0(x_pad, W1_0, W2_0, pv0)

    for w1, w2, pv in ((W1_1, W2_1, pv1), (W1_2, W2_2, pv2)):
        xm = _segmax(h, cl_pad, bounds, neg_init=False)
        g = _gather(xm, cl_pad)
        h = _layer(h, g, w1[:H, :], w1[H:, :], w2, pv)

    xm = _segmax(h, cl_pad, bounds, neg_init=False)
    g = _gather(xm, cl_pad)
    y = _finalpre(h, g, Wl[:H, :], Wl[H:, :], pvl)
    z = _segmax(y, cl_pad, bounds, neg_init=True)
    return _norm(z)[:C]


# speedup vs baseline: 1.0633x; 1.0633x over previous
import functools

import jax
import jax.numpy as jnp
from jax import lax
from jax.experimental import pallas as pl
from jax.experimental.pallas import tpu as pltpu
from jax.experimental.pallas import tpu_sc as plsc

N = 100000
IN_CH = 128
H = 64
C = 10000

NC = 2
NS = 16
NW = NC * NS

N_PAD = 102400
SGH = 320
CPT = 320
C_PAD = CPT * NW
SCH = 384
RB = 512
NEG = -3.0e38


def _bounds_body(cl_ref, out_ref):
    arr = cl_ref[...]
    acc = jnp.zeros((8, 128), jnp.int32)
    pos = lax.broadcasted_iota(jnp.int32, (8, 128), 0) * 128 + \
        lax.broadcasted_iota(jnp.int32, (8, 128), 1)
    for t in range(NW + 1):
        thr = min(CPT * t, C)
        cnt = jnp.sum((arr < thr).astype(jnp.int32))
        acc = jnp.where(pos == t, cnt, acc)
    out_ref[...] = acc


def _compute_bounds(cl2d):
    return pl.pallas_call(
        _bounds_body,
        out_shape=jax.ShapeDtypeStruct((8, 128), jnp.int32),
    )(cl2d)


def _ln(u, g, b):
    mu = jnp.mean(u, axis=-1, keepdims=True)
    var = jnp.mean((u - mu) ** 2, axis=-1, keepdims=True)
    return (u - mu) * lax.rsqrt(var + 1e-5) * g + b


def _mlp_tail(u, pv, w2_ref):
    u = jnp.maximum(_ln(u, pv[1:2, :], pv[2:3, :]), 0.0)
    v = jnp.dot(u, w2_ref[...], preferred_element_type=jnp.float32) + pv[3:4, :]
    return jnp.maximum(_ln(v, pv[4:5, :], pv[5:6, :]), 0.0)


def _mlp0_body(x_ref, w1_ref, w2_ref, pv_ref, out_ref):
    pv = pv_ref[...]
    u = jnp.dot(x_ref[...], w1_ref[...], preferred_element_type=jnp.float32)
    out_ref[...] = _mlp_tail(u + pv[0:1, :], pv, w2_ref)


def _mlp0(x, w1, w2, pv):
    return pl.pallas_call(
        _mlp0_body,
        grid=(N_PAD // RB,),
        in_specs=[
            pl.BlockSpec((RB, IN_CH), lambda i: (i, 0)),
            pl.BlockSpec((IN_CH, H), lambda i: (0, 0)),
            pl.BlockSpec((H, H), lambda i: (0, 0)),
            pl.BlockSpec((8, H), lambda i: (0, 0)),
        ],
        out_specs=pl.BlockSpec((RB, H), lambda i: (i, 0)),
        out_shape=jax.ShapeDtypeStruct((N_PAD, H), jnp.float32),
    )(x, w1, w2, pv)


def _layer_body(h_ref, g_ref, w1t_ref, w1b_ref, w2_ref, pv_ref, out_ref):
    pv = pv_ref[...]
    u = jnp.dot(h_ref[...], w1t_ref[...], preferred_element_type=jnp.float32)
    u = u + jnp.dot(g_ref[:, :H], w1b_ref[...],
                    preferred_element_type=jnp.float32)
    out_ref[...] = _mlp_tail(u + pv[0:1, :], pv, w2_ref)


def _layer(h, g, w1t, w1b, w2, pv):
    return pl.pallas_call(
        _layer_body,
        grid=(N_PAD // RB,),
        in_specs=[
            pl.BlockSpec((RB, H), lambda i: (i, 0)),
            pl.BlockSpec((RB, 2 * H), lambda i: (i, 0)),
            pl.BlockSpec((H, H), lambda i: (0, 0)),
            pl.BlockSpec((H, H), lambda i: (0, 0)),
            pl.BlockSpec((H, H), lambda i: (0, 0)),
            pl.BlockSpec((8, H), lambda i: (0, 0)),
        ],
        out_specs=pl.BlockSpec((RB, H), lambda i: (i, 0)),
        out_shape=jax.ShapeDtypeStruct((N_PAD, H), jnp.float32),
    )(h, g, w1t, w1b, w2, pv)


def _finalpre_body(h_ref, g_ref, wt_ref, wb_ref, pv_ref, out_ref):
    u = jnp.dot(h_ref[...], wt_ref[...], preferred_element_type=jnp.float32)
    u = u + jnp.dot(g_ref[:, :H], wb_ref[...],
                    preferred_element_type=jnp.float32)
    out_ref[...] = u + pv_ref[0:1, :]


def _finalpre(h, g, wlt, wlb, pv):
    return pl.pallas_call(
        _finalpre_body,
        grid=(N_PAD // RB,),
        in_specs=[
            pl.BlockSpec((RB, H), lambda i: (i, 0)),
            pl.BlockSpec((RB, 2 * H), lambda i: (i, 0)),
            pl.BlockSpec((H, H), lambda i: (0, 0)),
            pl.BlockSpec((H, H), lambda i: (0, 0)),
            pl.BlockSpec((8, H), lambda i: (0, 0)),
        ],
        out_specs=pl.BlockSpec((RB, H), lambda i: (i, 0)),
        out_shape=jax.ShapeDtypeStruct((N_PAD, H), jnp.float32),
    )(h, g, wlt, wlb, pv)


def _norm_body(z_ref, out_ref):
    z = z_ref[...]
    s = jnp.sum(z * z, axis=-1, keepdims=True)
    out_ref[...] = z * lax.rsqrt(jnp.maximum(s, 1e-24))


def _norm(z):
    return pl.pallas_call(
        _norm_body,
        out_shape=jax.ShapeDtypeStruct((C_PAD, H), jnp.float32),
    )(z)


@functools.lru_cache(maxsize=None)
def _sc_mesh():
    return plsc.VectorSubcoreMesh(core_axis_name="c", subcore_axis_name="s",
                                  num_cores=NC, num_subcores=NS)


def _segmax_sc_body(neg_init, h_hbm, cl_hbm, bounds_hbm, xm_hbm,
                    bounds_s, cl_v, h_v, acc_v, sem):
    wid = lax.axis_index("s") * NC + lax.axis_index("c")
    c0 = wid * CPT
    init = NEG if neg_init else 0.0
    negv = jnp.full((16,), NEG, jnp.float32)
    zero16f = jnp.zeros((16,), jnp.float32)
    initv = jnp.full((16,), init, jnp.float32)

    pltpu.sync_copy(bounds_hbm.at[pl.ds(0, 64)], bounds_s)
    rs = bounds_s[pl.ds(wid, 16)][0]
    re = bounds_s[pl.ds(wid + 1, 16)][0]
    base = (rs // SCH) * SCH
    nchunks = (re - base + (SCH - 1)) // SCH

    def zero_body(r, _):
        for q in range(H // 16):
            acc_v[r, pl.ds(16 * q, 16)] = initv
        return 0
    lax.fori_loop(0, CPT, zero_body, 0)

    def chunk_body(k, _):
        start = base + k * SCH
        pltpu.sync_copy(h_hbm.at[pl.ds(start, SCH)], h_v)
        pltpu.sync_copy(cl_hbm.at[pl.ds(start, SCH)], cl_v.at[pl.ds(0, SCH)])
        lo = jnp.maximum(rs - start, 0)
        hi = jnp.minimum(re - start, SCH)

        def row_body(j, _):
            li = cl_v[pl.ds(j, 16)][0] - c0
            for q in range(H // 16):
                v = h_v[j, pl.ds(16 * q, 16)]
                a = acc_v[li, pl.ds(16 * q, 16)]
                acc_v[li, pl.ds(16 * q, 16)] = jnp.maximum(a, v)
            return 0
        lax.fori_loop(lo, hi, row_body, 0)
        return 0
    lax.fori_loop(0, nchunks, chunk_body, 0)

    if neg_init:
        def fix_body(r, _):
            for q in range(H // 16):
                a = acc_v[r, pl.ds(16 * q, 16)]
                acc_v[r, pl.ds(16 * q, 16)] = jnp.where(a <= negv, zero16f, a)
            return 0
        lax.fori_loop(0, CPT, fix_body, 0)

    pltpu.sync_copy(acc_v, xm_hbm.at[pl.ds(c0, CPT)])


def _segmax(h, cl, bounds, neg_init):
    out = pl.kernel(
        functools.partial(_segmax_sc_body, neg_init),
        out_type=jax.ShapeDtypeStruct((C_PAD, H), jnp.float32),
        mesh=_sc_mesh(),
        compiler_params=pltpu.CompilerParams(needs_layout_passes=False),
        scratch_types=[
            pltpu.VMEM((64,), jnp.int32),
            pltpu.VMEM((SCH + 16,), jnp.int32),
            pltpu.VMEM((SCH, H), jnp.float32),
            pltpu.VMEM((CPT, H), jnp.float32),
            pltpu.SemaphoreType.DMA,
        ],
    )(h, cl, bounds)
    return out


def _segmax_gather_sc_body(h_hbm, cl_hbm, bounds_hbm, g_hbm,
                           bounds_s, cl_v, h_v, acc_v, gbuf, idx_v, sem):
    wid = lax.axis_index("s") * NC + lax.axis_index("c")
    c0 = wid * CPT
    iota16 = lax.iota(jnp.int32, 16)
    zerov = jnp.zeros((16,), jnp.float32)
    dump = jnp.full((16,), N + wid, jnp.int32)

    pltpu.sync_copy(bounds_hbm.at[pl.ds(0, 64)], bounds_s)
    rs = bounds_s[pl.ds(wid, 16)][0]
    re = bounds_s[pl.ds(wid + 1, 16)][0]
    base = (rs // SGH) * SGH
    nchunks = (re - base + (SGH - 1)) // SGH

    def zero_body(r, _):
        for q in range(H // 16):
            acc_v[r, pl.ds(16 * q, 16)] = zerov
        return 0
    lax.fori_loop(0, CPT, zero_body, 0)

    def chunk_body(k, _):
        start = base + k * SGH
        pltpu.sync_copy(h_hbm.at[pl.ds(start, SGH)], h_v)
        pltpu.sync_copy(cl_hbm.at[pl.ds(start, SGH)], cl_v.at[pl.ds(0, SGH)])
        lo = jnp.maximum(rs - start, 0)
        hi = jnp.minimum(re - start, SGH)

        def row_body(j, _):
            li = cl_v[pl.ds(j, 16)][0] - c0
            for q in range(H // 16):
                v = h_v[j, pl.ds(16 * q, 16)]
                a = acc_v[li, pl.ds(16 * q, 16)]
                acc_v[li, pl.ds(16 * q, 16)] = jnp.maximum(a, v)
            return 0
        lax.fori_loop(lo, hi, row_body, 0)
        return 0
    lax.fori_loop(0, nchunks, chunk_body, 0)

    def out_body(k, _):
        start = base + k * SGH
        pltpu.sync_copy(cl_hbm.at[pl.ds(start, SGH)], cl_v.at[pl.ds(0, SGH)])
        lo = jnp.maximum(rs - start, 0)
        hi = jnp.minimum(re - start, SGH)

        def idx_body(g, _):
            rvec = jnp.full((16,), start + g * 16, jnp.int32) + iota16
            valid = (rvec >= jnp.full((16,), rs, jnp.int32)) & \
                (rvec < jnp.full((16,), re, jnp.int32))
            idx_v[pl.ds(g * 16, 16)] = jnp.where(valid, rvec, dump)
            return 0
        lax.fori_loop(0, SGH // 16, idx_body, 0)

        def row_body(j, _):
            li = cl_v[pl.ds(j, 16)][0] - c0
            for q in range(H // 16):
                gbuf[j, pl.ds(16 * q, 16)] = acc_v[li, pl.ds(16 * q, 16)]
            return 0
        lax.fori_loop(lo, hi, row_body, 0)

        pltpu.sync_copy(gbuf, g_hbm.at[idx_v])
        return 0
    lax.fori_loop(0, nchunks, out_body, 0)


def _segmax_gather(h, cl, bounds):
    return pl.kernel(
        _segmax_gather_sc_body,
        out_type=jax.ShapeDtypeStruct((N_PAD, 2 * H), jnp.float32),
        mesh=_sc_mesh(),
        compiler_params=pltpu.CompilerParams(needs_layout_passes=False),
        scratch_types=[
            pltpu.VMEM((64,), jnp.int32),
            pltpu.VMEM((SGH + 16,), jnp.int32),
            pltpu.VMEM((SGH, H), jnp.float32),
            pltpu.VMEM((CPT, H), jnp.float32),
            pltpu.VMEM((SGH, 2 * H), jnp.float32),
            pltpu.VMEM((SGH,), jnp.int32),
            pltpu.SemaphoreType.DMA,
        ],
    )(h, cl, bounds)


def kernel(x, cluster, W1_0, b1_0, g1_0, be1_0, W2_0, b2_0, g2_0, be2_0,
           W1_1, b1_1, g1_1, be1_1, W2_1, b2_1, g2_1, be2_1,
           W1_2, b1_2, g1_2, be1_2, W2_2, b2_2, g2_2, be2_2, Wl, bl):
    x_pad = jnp.pad(x, ((0, N_PAD - N), (0, 0)))
    cl_pad = jnp.pad(cluster, (0, N_PAD - N), constant_values=C)
    cl2d = cl_pad.reshape(N_PAD // 128, 128)
    bounds = _compute_bounds(cl2d).reshape(-1)

    zeros = jnp.zeros((H,), jnp.float32)

    def pvec(rows):
        rows = list(rows) + [zeros] * (8 - len(rows))
        return jnp.stack(rows)

    pv0 = pvec([b1_0, g1_0, be1_0, b2_0, g2_0, be2_0])
    pv1 = pvec([b1_1, g1_1, be1_1, b2_1, g2_1, be2_1])
    pv2 = pvec([b1_2, g1_2, be1_2, b2_2, g2_2, be2_2])
    pvl = pvec([bl])

    h = _mlp0(x_pad, W1_0, W2_0, pv0)

    for w1, w2, pv in ((W1_1, W2_1, pv1), (W1_2, W2_2, pv2)):
        g = _segmax_gather(h, cl_pad, bounds)
        h = _layer(h, g, w1[:H, :], w1[H:, :], w2, pv)

    g = _segmax_gather(h, cl_pad, bounds)
    y = _finalpre(h, g, Wl[:H, :], Wl[H:, :], pvl)
    z = _segmax(y, cl_pad, bounds, neg_init=True)
    return _norm(z)[:C]


# speedup vs baseline: 1.2042x; 1.1325x over previous
import functools

import jax
import jax.numpy as jnp
from jax import lax
from jax.experimental import pallas as pl
from jax.experimental.pallas import tpu as pltpu
from jax.experimental.pallas import tpu_sc as plsc

N = 100000
IN_CH = 128
H = 64
C = 10000

NC = 2
NS = 16
NW = NC * NS

N_PAD = 102400
SGH = 320
CPT = 320
C_PAD = CPT * NW
SCH = 384
RB = 512
NEG = -3.0e38


def _bounds_body(cl_ref, out_ref):
    arr = cl_ref[...]
    acc = jnp.zeros((8, 128), jnp.int32)
    pos = lax.broadcasted_iota(jnp.int32, (8, 128), 0) * 128 + \
        lax.broadcasted_iota(jnp.int32, (8, 128), 1)
    for t in range(NW + 1):
        thr = min(CPT * t, C)
        cnt = jnp.sum((arr < thr).astype(jnp.int32))
        acc = jnp.where(pos == t, cnt, acc)
    out_ref[...] = acc


def _compute_bounds(cl2d):
    return pl.pallas_call(
        _bounds_body,
        out_shape=jax.ShapeDtypeStruct((8, 128), jnp.int32),
    )(cl2d)


def _ln(u, g, b):
    mu = jnp.mean(u, axis=-1, keepdims=True)
    var = jnp.mean((u - mu) ** 2, axis=-1, keepdims=True)
    return (u - mu) * lax.rsqrt(var + 1e-5) * g + b


def _mlp_tail(u, pv, w2_ref):
    u = jnp.maximum(_ln(u, pv[1:2, :], pv[2:3, :]), 0.0)
    v = jnp.dot(u, w2_ref[...], preferred_element_type=jnp.float32) + pv[3:4, :]
    return jnp.maximum(_ln(v, pv[4:5, :], pv[5:6, :]), 0.0)


def _mlp0_body(x_ref, w1_ref, w2_ref, pv_ref, out_ref):
    pv = pv_ref[...]
    u = jnp.dot(x_ref[...], w1_ref[...], preferred_element_type=jnp.float32)
    out_ref[...] = _mlp_tail(u + pv[0:1, :], pv, w2_ref)


def _mlp0(x, w1, w2, pv):
    return pl.pallas_call(
        _mlp0_body,
        grid=(N_PAD // RB,),
        in_specs=[
            pl.BlockSpec((RB, IN_CH), lambda i: (i, 0)),
            pl.BlockSpec((IN_CH, H), lambda i: (0, 0)),
            pl.BlockSpec((H, H), lambda i: (0, 0)),
            pl.BlockSpec((8, H), lambda i: (0, 0)),
        ],
        out_specs=pl.BlockSpec((RB, H), lambda i: (i, 0)),
        out_shape=jax.ShapeDtypeStruct((N_PAD, H), jnp.float32),
    )(x, w1, w2, pv)


def _layer_body(h_ref, g_ref, w1t_ref, w1b_ref, w2_ref, pv_ref, out_ref):
    pv = pv_ref[...]
    u = jnp.dot(h_ref[...], w1t_ref[...], preferred_element_type=jnp.float32)
    u = u + jnp.dot(g_ref[:, :H], w1b_ref[...],
                    preferred_element_type=jnp.float32)
    out_ref[...] = _mlp_tail(u + pv[0:1, :], pv, w2_ref)


def _layer(h, g, w1t, w1b, w2, pv):
    return pl.pallas_call(
        _layer_body,
        grid=(N_PAD // RB,),
        in_specs=[
            pl.BlockSpec((RB, H), lambda i: (i, 0)),
            pl.BlockSpec((RB, 2 * H), lambda i: (i, 0)),
            pl.BlockSpec((H, H), lambda i: (0, 0)),
            pl.BlockSpec((H, H), lambda i: (0, 0)),
            pl.BlockSpec((H, H), lambda i: (0, 0)),
            pl.BlockSpec((8, H), lambda i: (0, 0)),
        ],
        out_specs=pl.BlockSpec((RB, H), lambda i: (i, 0)),
        out_shape=jax.ShapeDtypeStruct((N_PAD, H), jnp.float32),
    )(h, g, w1t, w1b, w2, pv)


def _finalpre_body(h_ref, g_ref, wt_ref, wb_ref, pv_ref, out_ref):
    u = jnp.dot(h_ref[...], wt_ref[...], preferred_element_type=jnp.float32)
    u = u + jnp.dot(g_ref[:, :H], wb_ref[...],
                    preferred_element_type=jnp.float32)
    out_ref[...] = u + pv_ref[0:1, :]


def _finalpre(h, g, wlt, wlb, pv):
    return pl.pallas_call(
        _finalpre_body,
        grid=(N_PAD // RB,),
        in_specs=[
            pl.BlockSpec((RB, H), lambda i: (i, 0)),
            pl.BlockSpec((RB, 2 * H), lambda i: (i, 0)),
            pl.BlockSpec((H, H), lambda i: (0, 0)),
            pl.BlockSpec((H, H), lambda i: (0, 0)),
            pl.BlockSpec((8, H), lambda i: (0, 0)),
        ],
        out_specs=pl.BlockSpec((RB, H), lambda i: (i, 0)),
        out_shape=jax.ShapeDtypeStruct((N_PAD, H), jnp.float32),
    )(h, g, wlt, wlb, pv)


def _norm_body(z_ref, out_ref):
    z = z_ref[...]
    s = jnp.sum(z * z, axis=-1, keepdims=True)
    out_ref[...] = z * lax.rsqrt(jnp.maximum(s, 1e-24))


def _norm(z):
    return pl.pallas_call(
        _norm_body,
        out_shape=jax.ShapeDtypeStruct((C_PAD, H), jnp.float32),
    )(z)


@functools.lru_cache(maxsize=None)
def _sc_mesh():
    return plsc.VectorSubcoreMesh(core_axis_name="c", subcore_axis_name="s",
                                  num_cores=NC, num_subcores=NS)


def _segmax_sc_body(neg_init, h_hbm, cl_hbm, bounds_hbm, xm_hbm,
                    bounds_s, cl_v, h_v, acc_v, sem):
    wid = lax.axis_index("s") * NC + lax.axis_index("c")
    c0 = wid * CPT
    init = NEG if neg_init else 0.0
    negv = jnp.full((16,), NEG, jnp.float32)
    zero16f = jnp.zeros((16,), jnp.float32)
    initv = jnp.full((16,), init, jnp.float32)

    pltpu.sync_copy(bounds_hbm.at[pl.ds(0, 64)], bounds_s)
    rs = bounds_s[pl.ds(wid, 16)][0]
    re = bounds_s[pl.ds(wid + 1, 16)][0]
    base = (rs // SCH) * SCH
    nchunks = (re - base + (SCH - 1)) // SCH

    def zero_body(r, _):
        for q in range(H // 16):
            acc_v[r, pl.ds(16 * q, 16)] = initv
        return 0
    lax.fori_loop(0, CPT, zero_body, 0)

    def chunk_body(k, _):
        start = base + k * SCH
        pltpu.sync_copy(h_hbm.at[pl.ds(start, SCH)], h_v)
        pltpu.sync_copy(cl_hbm.at[pl.ds(start, SCH)], cl_v.at[pl.ds(0, SCH)])
        lo = jnp.maximum(rs - start, 0)
        hi = jnp.minimum(re - start, SCH)

        def row_body(j, _):
            li = cl_v[pl.ds(j, 16)][0] - c0
            for q in range(H // 16):
                v = h_v[j, pl.ds(16 * q, 16)]
                a = acc_v[li, pl.ds(16 * q, 16)]
                acc_v[li, pl.ds(16 * q, 16)] = jnp.maximum(a, v)
            return 0
        lax.fori_loop(lo, hi, row_body, 0)
        return 0
    lax.fori_loop(0, nchunks, chunk_body, 0)

    if neg_init:
        def fix_body(r, _):
            for q in range(H // 16):
                a = acc_v[r, pl.ds(16 * q, 16)]
                acc_v[r, pl.ds(16 * q, 16)] = jnp.where(a <= negv, zero16f, a)
            return 0
        lax.fori_loop(0, CPT, fix_body, 0)

    pltpu.sync_copy(acc_v, xm_hbm.at[pl.ds(c0, CPT)])


def _segmax(h, cl, bounds, neg_init):
    out = pl.kernel(
        functools.partial(_segmax_sc_body, neg_init),
        out_type=jax.ShapeDtypeStruct((C_PAD, H), jnp.float32),
        mesh=_sc_mesh(),
        compiler_params=pltpu.CompilerParams(needs_layout_passes=False),
        scratch_types=[
            pltpu.VMEM((64,), jnp.int32),
            pltpu.VMEM((SCH + 16,), jnp.int32),
            pltpu.VMEM((SCH, H), jnp.float32),
            pltpu.VMEM((CPT, H), jnp.float32),
            pltpu.SemaphoreType.DMA,
        ],
    )(h, cl, bounds)
    return out


def _segmax_gather_sc_body(h_hbm, cl_hbm, bounds_hbm, g_hbm,
                           bounds_s, cl_v, h_v, acc_v, gbuf, idx_v, sem):
    wid = lax.axis_index("s") * NC + lax.axis_index("c")
    c0 = wid * CPT
    iota16 = lax.iota(jnp.int32, 16)
    zerov = jnp.zeros((16,), jnp.float32)
    dump = jnp.full((16,), N + wid, jnp.int32)

    pltpu.sync_copy(bounds_hbm.at[pl.ds(0, 64)], bounds_s)
    rs = bounds_s[pl.ds(wid, 16)][0]
    re = bounds_s[pl.ds(wid + 1, 16)][0]
    base = (rs // SGH) * SGH
    nchunks = (re - base + (SGH - 1)) // SGH

    def zero_body(r, _):
        for q in range(H // 16):
            acc_v[r, pl.ds(16 * q, 16)] = zerov
        return 0
    lax.fori_loop(0, CPT + 1, zero_body, 0)

    def chunk_body(k, _):
        start = base + k * SGH
        pltpu.sync_copy(h_hbm.at[pl.ds(start, SGH)], h_v)
        pltpu.sync_copy(cl_hbm.at[pl.ds(start, SGH)], cl_v.at[pl.ds(0, SGH)])

        def grp_body(g, _):
            j0 = g * 16
            r0 = start + j0
            cl16 = cl_v[pl.ds(j0, 16)] - c0
            for kk in range(16):
                rj = r0 + kk
                ok = (rj >= rs) & (rj < re)
                li = jnp.where(ok, cl16[kk], CPT)
                j = j0 + kk
                for q in range(H // 16):
                    v = h_v[j, pl.ds(16 * q, 16)]
                    a = acc_v[li, pl.ds(16 * q, 16)]
                    acc_v[li, pl.ds(16 * q, 16)] = jnp.maximum(a, v)
            return 0
        lax.fori_loop(0, SGH // 16, grp_body, 0)
        return 0
    lax.fori_loop(0, nchunks, chunk_body, 0)

    def out_body(k, _):
        start = base + k * SGH
        pltpu.sync_copy(cl_hbm.at[pl.ds(start, SGH)], cl_v.at[pl.ds(0, SGH)])

        def grp_body(g, _):
            j0 = g * 16
            rvec = jnp.full((16,), start + j0, jnp.int32) + iota16
            valid = (rvec >= jnp.full((16,), rs, jnp.int32)) & \
                (rvec < jnp.full((16,), re, jnp.int32))
            idx_v[pl.ds(j0, 16)] = jnp.where(valid, rvec, dump)
            cl16 = cl_v[pl.ds(j0, 16)] - c0
            r0 = start + j0
            for kk in range(16):
                rj = r0 + kk
                ok = (rj >= rs) & (rj < re)
                li = jnp.where(ok, cl16[kk], CPT)
                j = j0 + kk
                for q in range(H // 16):
                    gbuf[j, pl.ds(16 * q, 16)] = acc_v[li, pl.ds(16 * q, 16)]
            return 0
        lax.fori_loop(0, SGH // 16, grp_body, 0)

        pltpu.sync_copy(gbuf, g_hbm.at[idx_v])
        return 0
    lax.fori_loop(0, nchunks, out_body, 0)


def _segmax_gather(h, cl, bounds):
    return pl.kernel(
        _segmax_gather_sc_body,
        out_type=jax.ShapeDtypeStruct((N_PAD, 2 * H), jnp.float32),
        mesh=_sc_mesh(),
        compiler_params=pltpu.CompilerParams(needs_layout_passes=False),
        scratch_types=[
            pltpu.VMEM((64,), jnp.int32),
            pltpu.VMEM((SGH + 16,), jnp.int32),
            pltpu.VMEM((SGH, H), jnp.float32),
            pltpu.VMEM((CPT + 1, H), jnp.float32),
            pltpu.VMEM((SGH, 2 * H), jnp.float32),
            pltpu.VMEM((SGH,), jnp.int32),
            pltpu.SemaphoreType.DMA,
        ],
    )(h, cl, bounds)


def kernel(x, cluster, W1_0, b1_0, g1_0, be1_0, W2_0, b2_0, g2_0, be2_0,
           W1_1, b1_1, g1_1, be1_1, W2_1, b2_1, g2_1, be2_1,
           W1_2, b1_2, g1_2, be1_2, W2_2, b2_2, g2_2, be2_2, Wl, bl):
    x_pad = jnp.pad(x, ((0, N_PAD - N), (0, 0)))
    cl_pad = jnp.pad(cluster, (0, N_PAD - N), constant_values=C)
    cl2d = cl_pad.reshape(N_PAD // 128, 128)
    bounds = _compute_bounds(cl2d).reshape(-1)

    zeros = jnp.zeros((H,), jnp.float32)

    def pvec(rows):
        rows = list(rows) + [zeros] * (8 - len(rows))
        return jnp.stack(rows)

    pv0 = pvec([b1_0, g1_0, be1_0, b2_0, g2_0, be2_0])
    pv1 = pvec([b1_1, g1_1, be1_1, b2_1, g2_1, be2_1])
    pv2 = pvec([b1_2, g1_2, be1_2, b2_2, g2_2, be2_2])
    pvl = pvec([bl])

    h = _mlp0(x_pad, W1_0, W2_0, pv0)

    for w1, w2, pv in ((W1_1, W2_1, pv1), (W1_2, W2_2, pv2)):
        g = _segmax_gather(h, cl_pad, bounds)
        h = _layer(h, g, w1[:H, :], w1[H:, :], w2, pv)

    g = _segmax_gather(h, cl_pad, bounds)
    y = _finalpre(h, g, Wl[:H, :], Wl[H:, :], pvl)
    z = _segmax(y, cl_pad, bounds, neg_init=True)
    return _norm(z)[:C]
